# baseline (speedup 1.0000x reference)
import functools

import jax
import jax.numpy as jnp
from jax import lax
from jax.experimental import pallas as pl
from jax.experimental.pallas import tpu as pltpu
from jax.experimental.pallas import tpu_sc as plsc

DEPTH = 1000
BATCH = 1024
GROUP = 26
ROWS = BATCH * GROUP
NC, NS, LANES = 2, 16, 16
NW = NC * NS
BPW = BATCH // NW
PAIRS = BPW // 2

_mesh = plsc.VectorSubcoreMesh(core_axis_name="c", subcore_axis_name="s")


@functools.partial(
    pl.kernel,
    out_type=jax.ShapeDtypeStruct((BATCH, GROUP, DEPTH), jnp.float32),
    mesh=_mesh,
    compiler_params=pltpu.CompilerParams(needs_layout_passes=False),
    scratch_types=[
        pltpu.VMEM((BPW * GROUP + LANES,), jnp.int32),
        pltpu.VMEM((1, GROUP, DEPTH), jnp.float32),
        pltpu.VMEM((1, GROUP, DEPTH), jnp.float32),
        pltpu.SemaphoreType.DMA,
        pltpu.SemaphoreType.DMA,
    ],
)
def _sc_onehot(idx_hbm, zeros_hbm, out_hbm, idx_v, buf0, buf1, sem0, sem1):
    wid = lax.axis_index("s") * NC + lax.axis_index("c")
    base_plane = wid * BPW
    pltpu.sync_copy(idx_hbm.at[pl.ds(base_plane * GROUP, BPW * GROUP)],
                    idx_v.at[pl.ds(0, BPW * GROUP)])
    pltpu.sync_copy(zeros_hbm, buf0)
    pltpu.sync_copy(zeros_hbm, buf1)

    ones = jnp.full((LANES,), 1.0, jnp.float32)
    zs = jnp.zeros((LANES,), jnp.float32)
    lane = lax.iota(jnp.int32, LANES)
    b_ids = jnp.zeros((LANES,), jnp.int32)
    tail_mask = lane < (GROUP - LANES)

    def plane_scatter(buf, c, val):
        d0 = idx_v[pl.ds(c * GROUP, LANES)]
        plsc.store_scatter(buf, [b_ids, lane, d0], val)
        d1 = idx_v[pl.ds(c * GROUP + LANES, LANES)]
        plsc.store_scatter(buf, [b_ids, lane + LANES, d1], val, mask=tail_mask)

    def dma(buf, c, sem):
        return pltpu.make_async_copy(
            buf, out_hbm.at[pl.ds(base_plane + c, 1)], sem)

    def body(i, _):
        c0 = 2 * i

        @pl.when(i > 0)
        def _():
            dma(buf0, c0, sem0).wait()
            plane_scatter(buf0, c0 - 2, zs)
            dma(buf1, c0, sem1).wait()
            plane_scatter(buf1, c0 - 1, zs)

        plane_scatter(buf0, c0, ones)
        dma(buf0, c0, sem0).start()
        plane_scatter(buf1, c0 + 1, ones)
        dma(buf1, c0 + 1, sem1).start()
        return _

    lax.fori_loop(0, PAIRS, body, None)
    dma(buf0, 0, sem0).wait()
    dma(buf1, 0, sem1).wait()


def kernel(inputs):
    flat_idx = inputs.reshape(ROWS)
    zeros = jnp.zeros((1, GROUP, DEPTH), jnp.float32)
    return _sc_onehot(flat_idx, zeros)


# speedup vs baseline: 5.9301x; 5.9301x over previous
import jax
import jax.numpy as jnp
from jax import lax
from jax.experimental import pallas as pl

DEPTH = 1000
BATCH = 1024
GROUP = 26
BLOCK_D = 40


def _onehot_body(idxt_ref, out_ref):
    idxt = idxt_ref[...]
    d0 = pl.program_id(0) * BLOCK_D
    dio = d0 + lax.broadcasted_iota(jnp.int32, (GROUP, BLOCK_D, BATCH), 1)
    out_ref[...] = (idxt[:, None, :] == dio).astype(jnp.float32)


def kernel(inputs):
    idxt = inputs.T
    out_t = pl.pallas_call(
        _onehot_body,
        grid=(DEPTH // BLOCK_D,),
        in_specs=[pl.BlockSpec((GROUP, BATCH), lambda i: (0, 0))],
        out_specs=pl.BlockSpec((GROUP, BLOCK_D, BATCH), lambda i: (0, i, 0)),
        out_shape=jax.ShapeDtypeStruct((GROUP, DEPTH, BATCH), jnp.float32),
    )(idxt)
    return out_t.transpose(2, 0, 1)
